# baseline (speedup 1.0000x reference)
import functools

import jax
import jax.numpy as jnp
import numpy as np
from jax.experimental import pallas as pl
from jax.experimental.pallas import tpu as pltpu

_TILE = 11136


def _fused_kernel(cu_ref, col_ref, xt_ref, w_enc_ref, b_enc_ref,
                  w_f_ref, b_f_ref, w_dect_ref, b_dec_ref, w_vt_ref, b_v_ref,
                  out_ref, val_ref, acc_ref, *, num_rows, num_steps):
    i = pl.program_id(0)

    @pl.when(i == 0)
    def _init():
        acc_ref[...] = jnp.zeros_like(acc_ref)

    b = cu_ref.shape[1] - 1
    tile = xt_ref.shape[1]

    starts_row = cu_ref[:, 0:b]
    ends_row = cu_ref[:, 1:b + 1]
    cu_col = cu_ref[...].reshape(b + 1, 1)
    starts = cu_col[0:b, :]
    ends = cu_col[1:b + 1, :]
    lo = jnp.concatenate([starts, starts], axis=0) - i * tile
    hi = jnp.concatenate([ends, starts + 1], axis=0) - i * tile

    w_enc = w_enc_ref[...].astype(jnp.bfloat16)
    b_enc_col = b_enc_ref[...].reshape(b_enc_ref.shape[1], 1).astype(jnp.bfloat16)
    col = col_ref[...]

    def accumulate(xt):
        z = jax.lax.dot_general(w_enc, xt.astype(jnp.bfloat16),
                                (((0,), (0,)), ((), ())),
                                preferred_element_type=jnp.float32)
        enc_t = jax.nn.relu(z.astype(jnp.bfloat16) + b_enc_col)
        oh = ((col >= lo) & (col < hi)).astype(jnp.bfloat16)
        acc_ref[...] += jax.lax.dot_general(
            enc_t, oh, (((1,), (1,)), ((), ())),
            preferred_element_type=jnp.float32)

    @pl.when(i < num_steps - 1)
    def _main():
        accumulate(xt_ref[...])

    @pl.when(i == num_steps - 1)
    def _last():
        valid = col + i * tile < num_rows
        accumulate(jnp.where(valid, xt_ref[...], 0.0))

        seg_sum_t = acc_ref[:, 0:b]
        agent_t = acc_ref[:, b:2 * b]
        neigh_sum_t = seg_sum_t - agent_t
        ncount = (ends_row - starts_row - 1).astype(jnp.float32)
        denom = jnp.maximum(ncount, 1.0)
        neigh_mean_t = jnp.where(ncount > 0.0, neigh_sum_t / denom, 0.0)
        f_in_t = jnp.concatenate([agent_t, neigh_mean_t], axis=0)
        hidden_t = jax.nn.relu(
            jax.lax.dot_general(w_f_ref[...], f_in_t,
                                (((0,), (0,)), ((), ())),
                                preferred_element_type=jnp.float32)
            + b_f_ref[...].reshape(w_f_ref.shape[1], 1)
        )
        out_ref[...] = (
            jax.lax.dot_general(hidden_t, w_dect_ref[...],
                                (((0,), (1,)), ((), ())),
                                preferred_element_type=jnp.float32)
            + b_dec_ref[...]
        )
        val_ref[...] = (
            jnp.sum(w_vt_ref[...].reshape(hidden_t.shape[0], 1) * hidden_t,
                    axis=0, keepdims=True)
            + b_v_ref[...]
        )


@jax.jit
def kernel(flat, cu_seqlens, segment_ids, W_enc, b_enc, W_f, b_f, W_dec, b_dec, W_v, b_v):
    del segment_ids
    t, e = flat.shape
    bsz = cu_seqlens.shape[0] - 1
    h = W_f.shape[1]
    n_out = W_dec.shape[1]
    num_steps = pl.cdiv(t, _TILE)
    col = jnp.asarray(np.arange(_TILE, dtype=np.int32).reshape(1, _TILE))

    full = lambda shape: pl.BlockSpec(shape, lambda i: (0,) * len(shape))
    out, value_t = pl.pallas_call(
        functools.partial(_fused_kernel, num_rows=t, num_steps=num_steps),
        grid=(num_steps,),
        in_specs=[
            full((1, bsz + 1)),
            full((1, _TILE)),
            pl.BlockSpec((e, _TILE), lambda i: (0, i)),
            full((e, e)),
            full((1, e)),
            full((2 * e, h)),
            full((1, h)),
            full((n_out, h)),
            full((1, n_out)),
            full((1, h)),
            full((1, 1)),
        ],
        out_specs=[
            full((bsz, n_out)),
            full((1, bsz)),
        ],
        out_shape=[
            jax.ShapeDtypeStruct((bsz, n_out), jnp.float32),
            jax.ShapeDtypeStruct((1, bsz), jnp.float32),
        ],
        scratch_shapes=[
            pltpu.VMEM((e, 2 * bsz), jnp.float32),
        ],
    )(
        cu_seqlens.reshape(1, bsz + 1), col, flat.T,
        W_enc, b_enc.reshape(1, e),
        W_f, b_f.reshape(1, h),
        W_dec.T, b_dec.reshape(1, n_out),
        W_v.T, b_v.reshape(1, 1),
    )
    return (out, value_t.T)


# speedup vs baseline: 1.1013x; 1.1013x over previous
import functools

import jax
import jax.numpy as jnp
import numpy as np
from jax.experimental import pallas as pl
from jax.experimental.pallas import tpu as pltpu

_TILE = 16640


def _fused_kernel(cu_ref, col_ref, xt_ref, w_enc_ref, b_enc_ref,
                  w_f_ref, b_f_ref, w_dect_ref, b_dec_ref, w_vt_ref, b_v_ref,
                  out_ref, val_ref, acc_ref, *, num_rows, num_steps):
    i = pl.program_id(0)

    @pl.when(i == 0)
    def _init():
        acc_ref[...] = jnp.zeros_like(acc_ref)

    b = cu_ref.shape[1] - 1
    tile = xt_ref.shape[1]

    starts_row = cu_ref[:, 0:b]
    ends_row = cu_ref[:, 1:b + 1]
    cu_col = cu_ref[...].reshape(b + 1, 1)
    starts = cu_col[0:b, :]
    ends = cu_col[1:b + 1, :]
    lo = jnp.concatenate([starts, starts], axis=0) - i * tile
    hi = jnp.concatenate([ends, starts + 1], axis=0) - i * tile

    w_enc = w_enc_ref[...].astype(jnp.bfloat16)
    b_enc_col = b_enc_ref[...].reshape(b_enc_ref.shape[1], 1).astype(jnp.bfloat16)
    col = col_ref[...]

    def accumulate(xt):
        z = jax.lax.dot_general(w_enc, xt.astype(jnp.bfloat16),
                                (((0,), (0,)), ((), ())),
                                preferred_element_type=jnp.float32)
        enc_t = jax.nn.relu(z.astype(jnp.bfloat16) + b_enc_col)
        oh = ((col >= lo) & (col < hi)).astype(jnp.bfloat16)
        acc_ref[...] += jax.lax.dot_general(
            enc_t, oh, (((1,), (1,)), ((), ())),
            preferred_element_type=jnp.float32)

    @pl.when(i < num_steps - 1)
    def _main():
        accumulate(xt_ref[...])

    @pl.when(i == num_steps - 1)
    def _last():
        valid = col + i * tile < num_rows
        accumulate(jnp.where(valid, xt_ref[...], 0.0))

        seg_sum_t = acc_ref[:, 0:b]
        agent_t = acc_ref[:, b:2 * b]
        neigh_sum_t = seg_sum_t - agent_t
        ncount = (ends_row - starts_row - 1).astype(jnp.float32)
        denom = jnp.maximum(ncount, 1.0)
        neigh_mean_t = jnp.where(ncount > 0.0, neigh_sum_t / denom, 0.0)
        f_in_t = jnp.concatenate([agent_t, neigh_mean_t], axis=0)
        hidden_t = jax.nn.relu(
            jax.lax.dot_general(w_f_ref[...], f_in_t,
                                (((0,), (0,)), ((), ())),
                                preferred_element_type=jnp.float32)
            + b_f_ref[...].reshape(w_f_ref.shape[1], 1)
        )
        out_ref[...] = (
            jax.lax.dot_general(hidden_t, w_dect_ref[...],
                                (((0,), (1,)), ((), ())),
                                preferred_element_type=jnp.float32)
            + b_dec_ref[...]
        )
        val_ref[...] = (
            jnp.sum(w_vt_ref[...].reshape(hidden_t.shape[0], 1) * hidden_t,
                    axis=0, keepdims=True)
            + b_v_ref[...]
        )


@jax.jit
def kernel(flat, cu_seqlens, segment_ids, W_enc, b_enc, W_f, b_f, W_dec, b_dec, W_v, b_v):
    del segment_ids
    t, e = flat.shape
    bsz = cu_seqlens.shape[0] - 1
    h = W_f.shape[1]
    n_out = W_dec.shape[1]
    num_steps = pl.cdiv(t, _TILE)
    col = jnp.asarray(np.arange(_TILE, dtype=np.int32).reshape(1, _TILE))

    full = lambda shape: pl.BlockSpec(shape, lambda i: (0,) * len(shape))
    out, value_t = pl.pallas_call(
        functools.partial(_fused_kernel, num_rows=t, num_steps=num_steps),
        grid=(num_steps,),
        in_specs=[
            full((1, bsz + 1)),
            full((1, _TILE)),
            pl.BlockSpec((e, _TILE), lambda i: (0, i)),
            full((e, e)),
            full((1, e)),
            full((2 * e, h)),
            full((1, h)),
            full((n_out, h)),
            full((1, n_out)),
            full((1, h)),
            full((1, 1)),
        ],
        out_specs=[
            full((bsz, n_out)),
            full((1, bsz)),
        ],
        out_shape=[
            jax.ShapeDtypeStruct((bsz, n_out), jnp.float32),
            jax.ShapeDtypeStruct((1, bsz), jnp.float32),
        ],
        scratch_shapes=[
            pltpu.VMEM((e, 2 * bsz), jnp.float32),
        ],
    )(
        cu_seqlens.reshape(1, bsz + 1), col, flat.T,
        W_enc, b_enc.reshape(1, e),
        W_f, b_f.reshape(1, h),
        W_dec.T, b_dec.reshape(1, n_out),
        W_v.T, b_v.reshape(1, 1),
    )
    return (out, value_t.T)
